# Initial kernel scaffold; baseline (speedup 1.0000x reference)
import jax
import jax.numpy as jnp
from jax.experimental import pallas as pl


def kernel(node_features, edge_index, W1, b1, W2, b2):
    raise NotImplementedError("write your pallas kernel here")


# speedup vs baseline: 14.9226x; 14.9226x over previous
import functools

import jax
import jax.numpy as jnp
from jax import lax
from jax.experimental import pallas as pl
from jax.experimental.pallas import tpu as pltpu
from jax.experimental.pallas import tpu_sc as plsc

N = 10000
NP = 10240
E = 320000
IN_D = 128
HID = 64
OUT_D = 50
OUT_P = 64

NC = 2
NS = 16
NW = NC * NS
EPW = E // NW
CH = 80
NCHUNK = EPW // CH
RPW = NP // NS
ROW_BLK = 1024
GRID = NP // ROW_BLK

_mesh = plsc.VectorSubcoreMesh(core_axis_name="c", subcore_axis_name="s")
_sc_params = pltpu.CompilerParams(use_tc_tiling_on_sc=False)


def _zero_stage(stage, rows, width):
    z = jnp.zeros((16,), jnp.float32)

    def body(j, _):
        for k in range(width // 16):
            stage[j, k * 16:(k + 1) * 16] = z
        return 0

    lax.fori_loop(0, rows, body, 0)


@functools.partial(
    pl.kernel,
    out_type=jax.ShapeDtypeStruct((NC, NP, 16), jnp.float32),
    mesh=_mesh,
    compiler_params=_sc_params,
    scratch_types=[
        pltpu.VMEM((CH,), jnp.int32),
        pltpu.VMEM((CH, 16), jnp.float32),
        pltpu.VMEM((RPW, 16), jnp.float32),
        pltpu.VMEM_SHARED((NP, 16), jnp.float32),
    ],
)
def _deg_kernel(dst_hbm, out_hbm, didx, ones_v, stage, acc):
    cid = lax.axis_index("c")
    sid = lax.axis_index("s")
    wid = sid * NC + cid

    one = jnp.ones((16,), jnp.float32)

    def fill(j, _):
        ones_v[j, 0:16] = one
        return 0

    lax.fori_loop(0, CH, fill, 0)
    _zero_stage(stage, RPW, 16)
    pltpu.sync_copy(stage, acc.at[pl.ds(sid * RPW, RPW)])
    plsc.subcore_barrier()

    base = wid * EPW

    def body(i, _):
        pltpu.sync_copy(dst_hbm.at[pl.ds(base + i * CH, CH)], didx)
        pltpu.sync_copy(ones_v, acc.at[didx], add=True)
        return 0

    lax.fori_loop(0, NCHUNK, body, 0)
    plsc.subcore_barrier()
    pltpu.sync_copy(acc.at[pl.ds(sid * RPW, RPW)],
                    out_hbm.at[cid, pl.ds(sid * RPW, RPW)])


def _make_agg(D):
    @functools.partial(
        pl.kernel,
        out_type=jax.ShapeDtypeStruct((NC, NP, D), jnp.float32),
        mesh=_mesh,
        compiler_params=_sc_params,
        scratch_types=[
            pltpu.VMEM((CH,), jnp.int32),
            pltpu.VMEM((CH,), jnp.int32),
            pltpu.VMEM((CH, D), jnp.float32),
            pltpu.VMEM((RPW, D), jnp.float32),
            pltpu.VMEM_SHARED((NP, D), jnp.float32),
            pltpu.SemaphoreType.DMA,
        ],
    )
    def agg(u_hbm, src_hbm, dst_hbm, out_hbm, sidx, didx, rows, stage, acc, sem):
        cid = lax.axis_index("c")
        sid = lax.axis_index("s")
        wid = sid * NC + cid

        _zero_stage(stage, RPW, D)
        pltpu.sync_copy(stage, acc.at[pl.ds(sid * RPW, RPW)])
        plsc.subcore_barrier()

        base = wid * EPW

        def body(i, _):
            off = base + i * CH
            pltpu.sync_copy(src_hbm.at[pl.ds(off, CH)], sidx)
            pltpu.sync_copy(dst_hbm.at[pl.ds(off, CH)], didx)
            pltpu.async_copy(u_hbm.at[sidx], rows, sem).wait()
            pltpu.sync_copy(rows, acc.at[didx], add=True)
            return 0

        lax.fori_loop(0, NCHUNK, body, 0)
        plsc.subcore_barrier()
        pltpu.sync_copy(acc.at[pl.ds(sid * RPW, RPW)],
                        out_hbm.at[cid, pl.ds(sid * RPW, RPW)])

    return agg


_agg64 = _make_agg(HID)


def _dinv_block(degp_ref):
    deg = 1.0 + degp_ref[0] + degp_ref[1]
    return lax.rsqrt(deg)[:, 0:1]


def _mm1_body(x_ref, w_ref, degp_ref, u_ref):
    dinv = _dinv_block(degp_ref)
    u_ref[...] = jnp.dot(x_ref[...], w_ref[...],
                         preferred_element_type=jnp.float32) * dinv


def _mm2_body(aggp_ref, u1_ref, degp_ref, b1_ref, w2_ref, u2_ref):
    dinv = _dinv_block(degp_ref)
    tot = aggp_ref[0] + aggp_ref[1] + u1_ref[...]
    h = jnp.maximum(tot * dinv + b1_ref[...], 0.0)
    u2_ref[...] = jnp.dot(h, w2_ref[...],
                          preferred_element_type=jnp.float32) * dinv


def _fin_body(aggp_ref, u2_ref, degp_ref, b2_ref, o_ref):
    dinv = _dinv_block(degp_ref)
    o_ref[...] = (aggp_ref[0] + aggp_ref[1] + u2_ref[...]) * dinv + b2_ref[...]


def _degp_spec():
    return pl.BlockSpec((NC, ROW_BLK, 16), lambda i: (0, i, 0))


def _mm1(x, W1, degp):
    return pl.pallas_call(
        _mm1_body,
        grid=(GRID,),
        in_specs=[
            pl.BlockSpec((ROW_BLK, IN_D), lambda i: (i, 0)),
            pl.BlockSpec((IN_D, HID), lambda i: (0, 0)),
            _degp_spec(),
        ],
        out_specs=pl.BlockSpec((ROW_BLK, HID), lambda i: (i, 0)),
        out_shape=jax.ShapeDtypeStruct((NP, HID), jnp.float32),
    )(x, W1, degp)


def _mm2(aggp, u1, degp, b1, W2p):
    return pl.pallas_call(
        _mm2_body,
        grid=(GRID,),
        in_specs=[
            pl.BlockSpec((NC, ROW_BLK, HID), lambda i: (0, i, 0)),
            pl.BlockSpec((ROW_BLK, HID), lambda i: (i, 0)),
            _degp_spec(),
            pl.BlockSpec((1, HID), lambda i: (0, 0)),
            pl.BlockSpec((HID, OUT_P), lambda i: (0, 0)),
        ],
        out_specs=pl.BlockSpec((ROW_BLK, OUT_P), lambda i: (i, 0)),
        out_shape=jax.ShapeDtypeStruct((NP, OUT_P), jnp.float32),
    )(aggp, u1, degp, b1, W2p)


def _fin(aggp, u2, degp, b2p):
    return pl.pallas_call(
        _fin_body,
        grid=(GRID,),
        in_specs=[
            pl.BlockSpec((NC, ROW_BLK, OUT_P), lambda i: (0, i, 0)),
            pl.BlockSpec((ROW_BLK, OUT_P), lambda i: (i, 0)),
            _degp_spec(),
            pl.BlockSpec((1, OUT_P), lambda i: (0, 0)),
        ],
        out_specs=pl.BlockSpec((ROW_BLK, OUT_P), lambda i: (i, 0)),
        out_shape=jax.ShapeDtypeStruct((NP, OUT_P), jnp.float32),
    )(aggp, u2, degp, b2p)


def kernel(node_features, edge_index, W1, b1, W2, b2):
    src = edge_index[0].astype(jnp.int32)
    dst = edge_index[1].astype(jnp.int32)
    xp = jnp.pad(node_features, ((0, NP - N), (0, 0)))

    degp = _deg_kernel(dst)
    u1 = _mm1(xp, W1, degp)
    agg1 = _agg64(u1, src, dst)

    W2p = jnp.zeros((HID, OUT_P), jnp.float32).at[:, :OUT_D].set(W2)
    b2p = jnp.zeros((OUT_P,), jnp.float32).at[:OUT_D].set(b2)

    u2 = _mm2(agg1, u1, degp, b1.reshape(1, HID), W2p)
    agg2 = _agg64(u2, src, dst)
    out = _fin(agg2, u2, degp, b2p.reshape(1, OUT_P))
    return out[:N, :OUT_D]
